# baseline (speedup 1.0000x reference)
import functools

import jax
import jax.numpy as jnp
from jax import lax
from jax.experimental import pallas as pl
from jax.experimental.pallas import tpu as pltpu
from jax.experimental.pallas import tpu_sc as plsc

_NC = 2
_NS = 16
_NW = _NC * _NS
_D = 128
_SUP = 1024
_RPH = 2
_HALF = 256


def _chunks8(total, cap):
    out = []
    rem = total
    while rem > 0:
        c = min(cap, rem)
        out.append(c)
        rem -= c
    return out


@functools.lru_cache(maxsize=None)
def _make_spmm(n_pad, e_pad):
    ept = e_pad // _NW
    nsup = ept // _SUP
    n_per_tile = n_pad // _NS
    zchunks = _chunks8(n_per_tile, _HALF)
    mesh = plsc.VectorSubcoreMesh(core_axis_name="c", subcore_axis_name="s",
                                  num_cores=_NC, num_subcores=_NS)

    @functools.partial(
        pl.kernel,
        out_type=jax.ShapeDtypeStruct((_NC * n_pad, _D), jnp.float32),
        mesh=mesh,
        scratch_types=[
            pltpu.VMEM_SHARED((n_pad, _D), jnp.float32),
            pltpu.VMEM((2, 8, 128), jnp.int32),
            pltpu.VMEM((2, 8, 128), jnp.int32),
            pltpu.VMEM((2, 8, 128), jnp.float32),
            pltpu.VMEM((_HALF, _D), jnp.float32),
            pltpu.SemaphoreType.DMA,
            pltpu.SemaphoreType.DMA,
            pltpu.SemaphoreType.DMA,
            pltpu.SemaphoreType.DMA,
        ],
    )
    def spmm(cur_hbm, srcr_hbm, dstr_hbm, ewr_hbm, out_hbm,
             acc, src_v, dst_v, ew_v, rows_v, gsem, isem, ssem0, ssem1):
        cid = lax.axis_index("c")
        sid = lax.axis_index("s")
        wid = cid * _NS + sid
        qbase = wid * nsup
        node_off = sid * n_per_tile

        def zrow(r, _):
            for j in range(_D // 16):
                rows_v[r, pl.ds(j * 16, 16)] = jnp.zeros((16,), jnp.float32)
            return 0
        lax.fori_loop(0, _HALF, zrow, 0)
        off = 0
        for zc in zchunks:
            pltpu.sync_copy(rows_v.at[pl.ds(0, zc)],
                            acc.at[pl.ds(node_off + off, zc)])
            off += zc
        plsc.subcore_barrier()

        pltpu.sync_copy(srcr_hbm.at[qbase], src_v.at[0])
        pltpu.sync_copy(dstr_hbm.at[qbase], dst_v.at[0])
        pltpu.sync_copy(ewr_hbm.at[qbase], ew_v.at[0])
        pltpu.async_copy(cur_hbm.at[src_v.at[0, 0]],
                         rows_v.at[pl.ds(0, 128)], gsem)
        pltpu.async_copy(rows_v.at[pl.ds(128, 128)],
                         acc.at[dst_v.at[0, 0]], ssem1, add=True)

        def sup_body(s, _):
            s8 = lax.rem(s, 2)
            nxt = 1 - s8
            sn = qbase + jnp.minimum(s + 1, nsup - 1)
            dis = did = die = None
            for j in range(8):
                sl = (j % 2) * 128
                osl = 128 - sl
                psem = ssem1 if j % 2 == 0 else ssem0
                csem = ssem0 if j % 2 == 0 else ssem1
                pltpu.make_async_copy(cur_hbm.at[src_v.at[s8, j]],
                                      rows_v.at[pl.ds(sl, 128)], gsem).wait()
                pltpu.make_async_copy(cur_hbm.at[src_v.at[s8, j]],
                                      rows_v.at[pl.ds(osl, 128)], psem).wait()
                if j == 0:
                    dis = pltpu.async_copy(srcr_hbm.at[sn], src_v.at[nxt], isem)
                    did = pltpu.async_copy(dstr_hbm.at[sn], dst_v.at[nxt], isem)
                    die = pltpu.async_copy(ewr_hbm.at[sn], ew_v.at[nxt], isem)
                if j < 7:
                    pltpu.async_copy(cur_hbm.at[src_v.at[s8, j + 1]],
                                     rows_v.at[pl.ds(osl, 128)], gsem)
                else:
                    dis.wait()
                    did.wait()
                    die.wait()
                    pltpu.async_copy(cur_hbm.at[src_v.at[nxt, 0]],
                                     rows_v.at[pl.ds(0, 128)], gsem)

                def gbody(g, _, j=j, sl=sl, s8=s8):
                    wvec = ew_v[s8, j, pl.ds(g * 16, 16)]
                    for l in range(16):
                        eo = sl + g * 16 + l
                        w = wvec.at[jnp.full((16,), l, jnp.int32)].get(
                            mode="promise_in_bounds")
                        for q in range(_D // 16):
                            rows_v[eo, pl.ds(q * 16, 16)] = (
                                rows_v[eo, pl.ds(q * 16, 16)] * w)
                    return 0
                lax.fori_loop(0, 8, gbody, 0)

                pltpu.async_copy(rows_v.at[pl.ds(sl, 128)],
                                 acc.at[dst_v.at[s8, j]], csem, add=True)
            return 0
        lax.fori_loop(0, nsup, sup_body, 0)
        pltpu.make_async_copy(cur_hbm.at[src_v.at[0, 0]],
                              rows_v.at[pl.ds(0, 128)], gsem).wait()
        pltpu.make_async_copy(cur_hbm.at[src_v.at[0, 0]],
                              rows_v.at[pl.ds(128, 128)], ssem1).wait()
        plsc.subcore_barrier()

        out_row = cid * n_pad + node_off
        off = 0
        for zc in zchunks:
            pltpu.sync_copy(acc.at[pl.ds(node_off + off, zc)],
                            out_hbm.at[pl.ds(out_row + off, zc)])
            off += zc

    return spmm


def _dense_body(p_ref, h_ref, w0_ref, w1_ref, b_ref, c1_ref, t_ref):
    c1 = p_ref[0] + p_ref[1]
    c1_ref[...] = c1
    t_ref[...] = (jnp.dot(h_ref[...], w0_ref[...], preferred_element_type=jnp.float32)
                  + jnp.dot(c1, w1_ref[...], preferred_element_type=jnp.float32)
                  + b_ref[...])


def _finish_body(relu, t_ref, p_ref, w2_ref, o_ref):
    c2 = p_ref[0] + p_ref[1]
    o = t_ref[...] + jnp.dot(c2, w2_ref[...], preferred_element_type=jnp.float32)
    if relu:
        o = jnp.where(o >= 0, o, 0.01 * o)
    o_ref[...] = o


def _dense_call(p, h, w0, w1, b2d):
    n, d = h.shape
    return pl.pallas_call(
        _dense_body,
        out_shape=(jax.ShapeDtypeStruct((n, d), jnp.float32),
                   jax.ShapeDtypeStruct((n, d), jnp.float32)),
    )(p, h, w0, w1, b2d)


def _finish_call(t, p, w2, relu):
    n, d = t.shape
    return pl.pallas_call(
        functools.partial(_finish_body, relu),
        out_shape=jax.ShapeDtypeStruct((n, d), jnp.float32),
    )(t, p, w2)


def kernel(x, edge_index, edge_attr, W0_0, W0_1, W0_2, b0,
           W1_0, W1_1, W1_2, b1, W2_0, W2_1, W2_2):
    n, d = x.shape
    e = edge_index.shape[1]
    src = edge_index[0]
    dst = edge_index[1]

    n_pad = -(-n // (_NS * 8)) * (_NS * 8)
    ept = -(-e // (_NW * _SUP)) * _SUP
    e_pad = ept * _NW
    pad = e_pad - e
    rt8 = e_pad // (8 * 128)
    srcp = jnp.concatenate([src, jnp.zeros((pad,), jnp.int32)]).reshape(rt8, 8, 128)
    pad_dst = jnp.arange(pad, dtype=jnp.int32) % jnp.int32(n_pad)
    dstp = jnp.concatenate([dst, pad_dst]).reshape(rt8, 8, 128)
    ewp = jnp.concatenate([edge_attr, jnp.zeros((pad,), jnp.float32)]).reshape(rt8, 8, 128)

    spmm = _make_spmm(n_pad, e_pad)

    def prop(cur):
        return spmm(cur, srcp, dstp, ewp).reshape(_NC, n_pad, d)

    def layer(h, w0, w1, w2, b2d, relu):
        p1 = prop(h)
        c1, t = _dense_call(p1, h, w0, w1, b2d)
        p2 = prop(c1)
        return _finish_call(t, p2, w2, relu)

    h = jnp.pad(x, ((0, n_pad - n), (0, 0)))
    h = layer(h, W0_0, W0_1, W0_2, b0.reshape(1, d), True)
    h = layer(h, W1_0, W1_1, W1_2, b1.reshape(1, d), True)
    w2p = [jnp.pad(w, ((0, 0), (0, d - w.shape[1]))) for w in (W2_0, W2_1, W2_2)]
    h = layer(h, w2p[0], w2p[1], w2p[2], jnp.zeros((1, d), jnp.float32), False)
    return h[:n, :W2_0.shape[1]]


# speedup vs baseline: 1.1960x; 1.1960x over previous
import functools

import jax
import jax.numpy as jnp
from jax import lax
from jax.experimental import pallas as pl
from jax.experimental.pallas import tpu as pltpu
from jax.experimental.pallas import tpu_sc as plsc

_NC = 2
_NS = 16
_NW = _NC * _NS
_D = 128
_SUP = 1024
_RPH = 2
_HALF = 256


def _chunks8(total, cap):
    out = []
    rem = total
    while rem > 0:
        c = min(cap, rem)
        out.append(c)
        rem -= c
    return out


@functools.lru_cache(maxsize=None)
def _make_spmm(n_pad, e_pad):
    tot = e_pad // (_NS * _SUP)
    nsup0 = max(1, min(tot - 1, (tot * 4) // 5))
    nsup1 = tot - nsup0
    n_per_tile = n_pad // _NS
    zchunks = _chunks8(n_per_tile, _HALF)
    mesh = plsc.VectorSubcoreMesh(core_axis_name="c", subcore_axis_name="s",
                                  num_cores=_NC, num_subcores=_NS)

    @functools.partial(
        pl.kernel,
        out_type=jax.ShapeDtypeStruct((_NC * n_pad, _D), jnp.float32),
        mesh=mesh,
        scratch_types=[
            pltpu.VMEM_SHARED((n_pad, _D), jnp.float32),
            pltpu.VMEM((2, 8, 128), jnp.int32),
            pltpu.VMEM((2, 8, 128), jnp.int32),
            pltpu.VMEM((2, 8, 128), jnp.float32),
            pltpu.VMEM((_HALF, _D), jnp.float32),
            pltpu.SemaphoreType.DMA,
            pltpu.SemaphoreType.DMA,
            pltpu.SemaphoreType.DMA,
            pltpu.SemaphoreType.DMA,
        ],
    )
    def spmm(cur_hbm, srcr_hbm, dstr_hbm, ewr_hbm, out_hbm,
             acc, src_v, dst_v, ew_v, rows_v, gsem, isem, ssem0, ssem1):
        cid = lax.axis_index("c")
        sid = lax.axis_index("s")
        nsup = jnp.where(cid == 0, nsup0, nsup1)
        qbase = sid * tot + cid * nsup0
        node_off = sid * n_per_tile

        def zrow(r, _):
            for j in range(_D // 16):
                rows_v[r, pl.ds(j * 16, 16)] = jnp.zeros((16,), jnp.float32)
            return 0
        lax.fori_loop(0, _HALF, zrow, 0)
        off = 0
        for zc in zchunks:
            pltpu.sync_copy(rows_v.at[pl.ds(0, zc)],
                            acc.at[pl.ds(node_off + off, zc)])
            off += zc
        plsc.subcore_barrier()

        pltpu.sync_copy(srcr_hbm.at[qbase], src_v.at[0])
        pltpu.sync_copy(dstr_hbm.at[qbase], dst_v.at[0])
        pltpu.sync_copy(ewr_hbm.at[qbase], ew_v.at[0])
        pltpu.async_copy(cur_hbm.at[src_v.at[0, 0]],
                         rows_v.at[pl.ds(0, 128)], gsem)
        pltpu.async_copy(rows_v.at[pl.ds(128, 128)],
                         acc.at[dst_v.at[0, 0]], ssem1, add=True)

        def sup_body(s, _):
            s8 = lax.rem(s, 2)
            nxt = 1 - s8
            sn = qbase + jnp.minimum(s + 1, nsup - 1)
            dis = did = die = None
            for j in range(8):
                sl = (j % 2) * 128
                osl = 128 - sl
                psem = ssem1 if j % 2 == 0 else ssem0
                csem = ssem0 if j % 2 == 0 else ssem1
                pltpu.make_async_copy(cur_hbm.at[src_v.at[s8, j]],
                                      rows_v.at[pl.ds(sl, 128)], gsem).wait()
                pltpu.make_async_copy(cur_hbm.at[src_v.at[s8, j]],
                                      rows_v.at[pl.ds(osl, 128)], psem).wait()
                if j == 0:
                    dis = pltpu.async_copy(srcr_hbm.at[sn], src_v.at[nxt], isem)
                    did = pltpu.async_copy(dstr_hbm.at[sn], dst_v.at[nxt], isem)
                    die = pltpu.async_copy(ewr_hbm.at[sn], ew_v.at[nxt], isem)
                if j < 7:
                    pltpu.async_copy(cur_hbm.at[src_v.at[s8, j + 1]],
                                     rows_v.at[pl.ds(osl, 128)], gsem)
                else:
                    dis.wait()
                    did.wait()
                    die.wait()
                    pltpu.async_copy(cur_hbm.at[src_v.at[nxt, 0]],
                                     rows_v.at[pl.ds(0, 128)], gsem)

                def gbody(g, _, j=j, sl=sl, s8=s8):
                    wvec = ew_v[s8, j, pl.ds(g * 16, 16)]
                    for l in range(16):
                        eo = sl + g * 16 + l
                        w = wvec.at[jnp.full((16,), l, jnp.int32)].get(
                            mode="promise_in_bounds")
                        for q in range(_D // 16):
                            rows_v[eo, pl.ds(q * 16, 16)] = (
                                rows_v[eo, pl.ds(q * 16, 16)] * w)
                    return 0
                lax.fori_loop(0, 8, gbody, 0)

                pltpu.async_copy(rows_v.at[pl.ds(sl, 128)],
                                 acc.at[dst_v.at[s8, j]], csem, add=True)
            return 0
        lax.fori_loop(0, nsup, sup_body, 0)
        pltpu.make_async_copy(cur_hbm.at[src_v.at[0, 0]],
                              rows_v.at[pl.ds(0, 128)], gsem).wait()
        pltpu.make_async_copy(cur_hbm.at[src_v.at[0, 0]],
                              rows_v.at[pl.ds(128, 128)], ssem1).wait()
        plsc.subcore_barrier()

        out_row = cid * n_pad + node_off
        off = 0
        for zc in zchunks:
            pltpu.sync_copy(acc.at[pl.ds(node_off + off, zc)],
                            out_hbm.at[pl.ds(out_row + off, zc)])
            off += zc

    return spmm


def _dense_body(p_ref, h_ref, w0_ref, w1_ref, b_ref, c1_ref, t_ref):
    c1 = p_ref[0] + p_ref[1]
    c1_ref[...] = c1
    t_ref[...] = (jnp.dot(h_ref[...], w0_ref[...], preferred_element_type=jnp.float32)
                  + jnp.dot(c1, w1_ref[...], preferred_element_type=jnp.float32)
                  + b_ref[...])


def _finish_body(relu, t_ref, p_ref, w2_ref, o_ref):
    c2 = p_ref[0] + p_ref[1]
    o = t_ref[...] + jnp.dot(c2, w2_ref[...], preferred_element_type=jnp.float32)
    if relu:
        o = jnp.where(o >= 0, o, 0.01 * o)
    o_ref[...] = o


def _dense_call(p, h, w0, w1, b2d):
    n, d = h.shape
    return pl.pallas_call(
        _dense_body,
        out_shape=(jax.ShapeDtypeStruct((n, d), jnp.float32),
                   jax.ShapeDtypeStruct((n, d), jnp.float32)),
    )(p, h, w0, w1, b2d)


def _finish_call(t, p, w2, relu):
    n, d = t.shape
    return pl.pallas_call(
        functools.partial(_finish_body, relu),
        out_shape=jax.ShapeDtypeStruct((n, d), jnp.float32),
    )(t, p, w2)


def kernel(x, edge_index, edge_attr, W0_0, W0_1, W0_2, b0,
           W1_0, W1_1, W1_2, b1, W2_0, W2_1, W2_2):
    n, d = x.shape
    e = edge_index.shape[1]
    src = edge_index[0]
    dst = edge_index[1]

    n_pad = -(-n // (_NS * 8)) * (_NS * 8)
    ept = -(-e // (_NW * _SUP)) * _SUP
    e_pad = ept * _NW
    pad = e_pad - e
    rt8 = e_pad // (8 * 128)
    srcp = jnp.concatenate([src, jnp.zeros((pad,), jnp.int32)]).reshape(rt8, 8, 128)
    pad_dst = jnp.arange(pad, dtype=jnp.int32) % jnp.int32(n_pad)
    dstp = jnp.concatenate([dst, pad_dst]).reshape(rt8, 8, 128)
    ewp = jnp.concatenate([edge_attr, jnp.zeros((pad,), jnp.float32)]).reshape(rt8, 8, 128)

    spmm = _make_spmm(n_pad, e_pad)

    def prop(cur):
        return spmm(cur, srcp, dstp, ewp).reshape(_NC, n_pad, d)

    def layer(h, w0, w1, w2, b2d, relu):
        p1 = prop(h)
        c1, t = _dense_call(p1, h, w0, w1, b2d)
        p2 = prop(c1)
        return _finish_call(t, p2, w2, relu)

    h = jnp.pad(x, ((0, n_pad - n), (0, 0)))
    h = layer(h, W0_0, W0_1, W0_2, b0.reshape(1, d), True)
    h = layer(h, W1_0, W1_1, W1_2, b1.reshape(1, d), True)
    w2p = [jnp.pad(w, ((0, 0), (0, d - w.shape[1]))) for w in (W2_0, W2_1, W2_2)]
    h = layer(h, w2p[0], w2p[1], w2p[2], jnp.zeros((1, d), jnp.float32), False)
    return h[:n, :W2_0.shape[1]]


# speedup vs baseline: 1.2070x; 1.0092x over previous
import functools

import jax
import jax.numpy as jnp
from jax import lax
from jax.experimental import pallas as pl
from jax.experimental.pallas import tpu as pltpu
from jax.experimental.pallas import tpu_sc as plsc

_NC = 2
_NS = 16
_NW = _NC * _NS
_D = 128
_SUP = 1024
_RPH = 2
_HALF = 256


def _chunks8(total, cap):
    out = []
    rem = total
    while rem > 0:
        c = min(cap, rem)
        out.append(c)
        rem -= c
    return out


@functools.lru_cache(maxsize=None)
def _make_spmm(n_pad, e_pad):
    tot = e_pad // (_NS * _SUP)
    nsup0 = max(1, min(tot - 1, (tot * 7) // 10))
    nsup1 = tot - nsup0
    n_per_tile = n_pad // _NS
    zchunks = _chunks8(n_per_tile, _HALF)
    mesh = plsc.VectorSubcoreMesh(core_axis_name="c", subcore_axis_name="s",
                                  num_cores=_NC, num_subcores=_NS)

    @functools.partial(
        pl.kernel,
        out_type=jax.ShapeDtypeStruct((_NC * n_pad, _D), jnp.float32),
        mesh=mesh,
        scratch_types=[
            pltpu.VMEM_SHARED((n_pad, _D), jnp.float32),
            pltpu.VMEM((2, 8, 128), jnp.int32),
            pltpu.VMEM((2, 8, 128), jnp.int32),
            pltpu.VMEM((2, 8, 128), jnp.float32),
            pltpu.VMEM((_HALF, _D), jnp.float32),
            pltpu.SemaphoreType.DMA,
            pltpu.SemaphoreType.DMA,
            pltpu.SemaphoreType.DMA,
            pltpu.SemaphoreType.DMA,
        ],
    )
    def spmm(cur_hbm, srcr_hbm, dstr_hbm, ewr_hbm, out_hbm,
             acc, src_v, dst_v, ew_v, rows_v, gsem, isem, ssem0, ssem1):
        cid = lax.axis_index("c")
        sid = lax.axis_index("s")
        nsup = jnp.where(cid == 0, nsup0, nsup1)
        qbase = sid * tot + cid * nsup0
        node_off = sid * n_per_tile

        def zrow(r, _):
            for j in range(_D // 16):
                rows_v[r, pl.ds(j * 16, 16)] = jnp.zeros((16,), jnp.float32)
            return 0
        lax.fori_loop(0, _HALF, zrow, 0)
        off = 0
        for zc in zchunks:
            pltpu.sync_copy(rows_v.at[pl.ds(0, zc)],
                            acc.at[pl.ds(node_off + off, zc)])
            off += zc
        plsc.subcore_barrier()

        pltpu.sync_copy(srcr_hbm.at[qbase], src_v.at[0])
        pltpu.sync_copy(dstr_hbm.at[qbase], dst_v.at[0])
        pltpu.sync_copy(ewr_hbm.at[qbase], ew_v.at[0])
        pltpu.async_copy(cur_hbm.at[src_v.at[0, 0]],
                         rows_v.at[pl.ds(0, 128)], gsem)
        pltpu.async_copy(rows_v.at[pl.ds(128, 128)],
                         acc.at[dst_v.at[0, 0]], ssem1, add=True)

        def sup_body(s, _):
            s8 = lax.rem(s, 2)
            nxt = 1 - s8
            sn = qbase + jnp.minimum(s + 1, nsup - 1)
            dis = did = die = None
            for j in range(8):
                sl = (j % 2) * 128
                osl = 128 - sl
                psem = ssem1 if j % 2 == 0 else ssem0
                csem = ssem0 if j % 2 == 0 else ssem1
                pltpu.make_async_copy(cur_hbm.at[src_v.at[s8, j]],
                                      rows_v.at[pl.ds(sl, 128)], gsem).wait()
                pltpu.make_async_copy(cur_hbm.at[src_v.at[s8, j]],
                                      rows_v.at[pl.ds(osl, 128)], psem).wait()
                if j == 0:
                    dis = pltpu.async_copy(srcr_hbm.at[sn], src_v.at[nxt], isem)
                    did = pltpu.async_copy(dstr_hbm.at[sn], dst_v.at[nxt], isem)
                    die = pltpu.async_copy(ewr_hbm.at[sn], ew_v.at[nxt], isem)
                if j < 7:
                    pltpu.async_copy(cur_hbm.at[src_v.at[s8, j + 1]],
                                     rows_v.at[pl.ds(osl, 128)], gsem)
                else:
                    dis.wait()
                    did.wait()
                    die.wait()
                    pltpu.async_copy(cur_hbm.at[src_v.at[nxt, 0]],
                                     rows_v.at[pl.ds(0, 128)], gsem)

                def gbody(g, _, j=j, sl=sl, s8=s8):
                    wvec = ew_v[s8, j, pl.ds(g * 16, 16)]
                    for l in range(16):
                        eo = sl + g * 16 + l
                        w = wvec.at[jnp.full((16,), l, jnp.int32)].get(
                            mode="promise_in_bounds")
                        for q in range(_D // 16):
                            rows_v[eo, pl.ds(q * 16, 16)] = (
                                rows_v[eo, pl.ds(q * 16, 16)] * w)
                    return 0
                lax.fori_loop(0, 8, gbody, 0)

                pltpu.async_copy(rows_v.at[pl.ds(sl, 128)],
                                 acc.at[dst_v.at[s8, j]], csem, add=True)
            return 0
        lax.fori_loop(0, nsup, sup_body, 0)
        pltpu.make_async_copy(cur_hbm.at[src_v.at[0, 0]],
                              rows_v.at[pl.ds(0, 128)], gsem).wait()
        pltpu.make_async_copy(cur_hbm.at[src_v.at[0, 0]],
                              rows_v.at[pl.ds(128, 128)], ssem1).wait()
        plsc.subcore_barrier()

        out_row = cid * n_pad + node_off
        off = 0
        for zc in zchunks:
            pltpu.sync_copy(acc.at[pl.ds(node_off + off, zc)],
                            out_hbm.at[pl.ds(out_row + off, zc)])
            off += zc

    return spmm


def _dense_body(p_ref, h_ref, w0_ref, w1_ref, b_ref, c1_ref, t_ref):
    c1 = p_ref[0] + p_ref[1]
    c1_ref[...] = c1
    t_ref[...] = (jnp.dot(h_ref[...], w0_ref[...], preferred_element_type=jnp.float32)
                  + jnp.dot(c1, w1_ref[...], preferred_element_type=jnp.float32)
                  + b_ref[...])


def _finish_body(relu, t_ref, p_ref, w2_ref, o_ref):
    c2 = p_ref[0] + p_ref[1]
    o = t_ref[...] + jnp.dot(c2, w2_ref[...], preferred_element_type=jnp.float32)
    if relu:
        o = jnp.where(o >= 0, o, 0.01 * o)
    o_ref[...] = o


def _dense_call(p, h, w0, w1, b2d):
    n, d = h.shape
    return pl.pallas_call(
        _dense_body,
        out_shape=(jax.ShapeDtypeStruct((n, d), jnp.float32),
                   jax.ShapeDtypeStruct((n, d), jnp.float32)),
    )(p, h, w0, w1, b2d)


def _finish_call(t, p, w2, relu):
    n, d = t.shape
    return pl.pallas_call(
        functools.partial(_finish_body, relu),
        out_shape=jax.ShapeDtypeStruct((n, d), jnp.float32),
    )(t, p, w2)


def kernel(x, edge_index, edge_attr, W0_0, W0_1, W0_2, b0,
           W1_0, W1_1, W1_2, b1, W2_0, W2_1, W2_2):
    n, d = x.shape
    e = edge_index.shape[1]
    src = edge_index[0]
    dst = edge_index[1]

    n_pad = -(-n // (_NS * 8)) * (_NS * 8)
    ept = -(-e // (_NW * _SUP)) * _SUP
    e_pad = ept * _NW
    pad = e_pad - e
    rt8 = e_pad // (8 * 128)
    srcp = jnp.concatenate([src, jnp.zeros((pad,), jnp.int32)]).reshape(rt8, 8, 128)
    pad_dst = jnp.arange(pad, dtype=jnp.int32) % jnp.int32(n_pad)
    dstp = jnp.concatenate([dst, pad_dst]).reshape(rt8, 8, 128)
    ewp = jnp.concatenate([edge_attr, jnp.zeros((pad,), jnp.float32)]).reshape(rt8, 8, 128)

    spmm = _make_spmm(n_pad, e_pad)

    def prop(cur):
        return spmm(cur, srcp, dstp, ewp).reshape(_NC, n_pad, d)

    def layer(h, w0, w1, w2, b2d, relu):
        p1 = prop(h)
        c1, t = _dense_call(p1, h, w0, w1, b2d)
        p2 = prop(c1)
        return _finish_call(t, p2, w2, relu)

    h = jnp.pad(x, ((0, n_pad - n), (0, 0)))
    h = layer(h, W0_0, W0_1, W0_2, b0.reshape(1, d), True)
    h = layer(h, W1_0, W1_1, W1_2, b1.reshape(1, d), True)
    w2p = [jnp.pad(w, ((0, 0), (0, d - w.shape[1]))) for w in (W2_0, W2_1, W2_2)]
    h = layer(h, w2p[0], w2p[1], w2p[2], jnp.zeros((1, d), jnp.float32), False)
    return h[:n, :W2_0.shape[1]]


# speedup vs baseline: 1.2106x; 1.0029x over previous
import functools

import jax
import jax.numpy as jnp
from jax import lax
from jax.experimental import pallas as pl
from jax.experimental.pallas import tpu as pltpu
from jax.experimental.pallas import tpu_sc as plsc

_NC = 2
_NS = 16
_NW = _NC * _NS
_D = 128
_SUP = 1024
_RPH = 2
_HALF = 256


def _chunks8(total, cap):
    out = []
    rem = total
    while rem > 0:
        c = min(cap, rem)
        out.append(c)
        rem -= c
    return out


@functools.lru_cache(maxsize=None)
def _make_spmm(n_pad, e_pad):
    tot = e_pad // (_NS * _SUP)
    nsup0 = max(1, min(tot - 1, (tot * 7) // 10))
    nsup1 = tot - nsup0
    n_per_tile = n_pad // _NS
    zchunks = _chunks8(n_per_tile, _HALF)
    mesh = plsc.VectorSubcoreMesh(core_axis_name="c", subcore_axis_name="s",
                                  num_cores=_NC, num_subcores=_NS)

    @functools.partial(
        pl.kernel,
        out_type=jax.ShapeDtypeStruct((_NC * n_pad, _D), jnp.float32),
        mesh=mesh,
        scratch_types=[
            pltpu.VMEM_SHARED((n_pad, _D), jnp.float32),
            pltpu.VMEM((2, 8, 128), jnp.int32),
            pltpu.VMEM((2, 8, 128), jnp.int32),
            pltpu.VMEM((2, 8, 128), jnp.float32),
            pltpu.VMEM((_HALF, _D), jnp.float32),
            pltpu.SemaphoreType.DMA,
            pltpu.SemaphoreType.DMA,
            pltpu.SemaphoreType.DMA,
            pltpu.SemaphoreType.DMA,
        ],
    )
    def spmm(cur_hbm, srcr_hbm, dstr_hbm, ewr_hbm, out_hbm,
             acc, src_v, dst_v, ew_v, rows_v, gsem, isem, ssem0, ssem1):
        cid = lax.axis_index("c")
        sid = lax.axis_index("s")
        nsup = jnp.where(cid == 0, nsup0, nsup1)
        qbase = sid * tot + cid * nsup0
        node_off = sid * n_per_tile

        pis = pltpu.async_copy(srcr_hbm.at[qbase], src_v.at[0], isem)
        pid = pltpu.async_copy(dstr_hbm.at[qbase], dst_v.at[0], isem)
        pie = pltpu.async_copy(ewr_hbm.at[qbase], ew_v.at[0], isem)

        def zrow(r, _):
            for j in range(_D // 16):
                rows_v[r, pl.ds(j * 16, 16)] = jnp.zeros((16,), jnp.float32)
            return 0
        lax.fori_loop(0, _HALF, zrow, 0)
        zcs = []
        off = 0
        for zc in zchunks:
            zcs.append(pltpu.async_copy(rows_v.at[pl.ds(0, zc)],
                                        acc.at[pl.ds(node_off + off, zc)], gsem))
            off += zc
        for c in zcs:
            c.wait()
        plsc.subcore_barrier()

        pis.wait()
        pid.wait()
        pie.wait()
        pltpu.async_copy(cur_hbm.at[src_v.at[0, 0]],
                         rows_v.at[pl.ds(0, 128)], gsem)
        pltpu.async_copy(rows_v.at[pl.ds(128, 128)],
                         acc.at[dst_v.at[0, 0]], ssem1, add=True)

        def sup_body(s, _):
            s8 = lax.rem(s, 2)
            nxt = 1 - s8
            sn = qbase + jnp.minimum(s + 1, nsup - 1)
            dis = did = die = None
            for j in range(8):
                sl = (j % 2) * 128
                osl = 128 - sl
                psem = ssem1 if j % 2 == 0 else ssem0
                csem = ssem0 if j % 2 == 0 else ssem1
                pltpu.make_async_copy(cur_hbm.at[src_v.at[s8, j]],
                                      rows_v.at[pl.ds(sl, 128)], gsem).wait()
                pltpu.make_async_copy(cur_hbm.at[src_v.at[s8, j]],
                                      rows_v.at[pl.ds(osl, 128)], psem).wait()
                if j == 0:
                    dis = pltpu.async_copy(srcr_hbm.at[sn], src_v.at[nxt], isem)
                    did = pltpu.async_copy(dstr_hbm.at[sn], dst_v.at[nxt], isem)
                    die = pltpu.async_copy(ewr_hbm.at[sn], ew_v.at[nxt], isem)
                if j < 7:
                    pltpu.async_copy(cur_hbm.at[src_v.at[s8, j + 1]],
                                     rows_v.at[pl.ds(osl, 128)], gsem)
                else:
                    dis.wait()
                    did.wait()
                    die.wait()
                    pltpu.async_copy(cur_hbm.at[src_v.at[nxt, 0]],
                                     rows_v.at[pl.ds(0, 128)], gsem)

                def gbody(g, _, j=j, sl=sl, s8=s8):
                    wvec = ew_v[s8, j, pl.ds(g * 16, 16)]
                    for l in range(16):
                        eo = sl + g * 16 + l
                        w = wvec.at[jnp.full((16,), l, jnp.int32)].get(
                            mode="promise_in_bounds")
                        for q in range(_D // 16):
                            rows_v[eo, pl.ds(q * 16, 16)] = (
                                rows_v[eo, pl.ds(q * 16, 16)] * w)
                    return 0
                lax.fori_loop(0, 8, gbody, 0)

                pltpu.async_copy(rows_v.at[pl.ds(sl, 128)],
                                 acc.at[dst_v.at[s8, j]], csem, add=True)
            return 0
        lax.fori_loop(0, nsup, sup_body, 0)
        pltpu.make_async_copy(cur_hbm.at[src_v.at[0, 0]],
                              rows_v.at[pl.ds(0, 128)], gsem).wait()
        pltpu.make_async_copy(cur_hbm.at[src_v.at[0, 0]],
                              rows_v.at[pl.ds(128, 128)], ssem1).wait()
        plsc.subcore_barrier()

        out_row = cid * n_pad + node_off
        cps = []
        off = 0
        for zc in zchunks:
            cps.append(pltpu.async_copy(acc.at[pl.ds(node_off + off, zc)],
                                        out_hbm.at[pl.ds(out_row + off, zc)],
                                        gsem))
            off += zc
        for c in cps:
            c.wait()

    return spmm


def _dense_body(p_ref, h_ref, w0_ref, w1_ref, b_ref, c1_ref, t_ref):
    c1 = p_ref[0] + p_ref[1]
    c1_ref[...] = c1
    t_ref[...] = (jnp.dot(h_ref[...], w0_ref[...], preferred_element_type=jnp.float32)
                  + jnp.dot(c1, w1_ref[...], preferred_element_type=jnp.float32)
                  + b_ref[...])


def _finish_body(relu, t_ref, p_ref, w2_ref, o_ref):
    c2 = p_ref[0] + p_ref[1]
    o = t_ref[...] + jnp.dot(c2, w2_ref[...], preferred_element_type=jnp.float32)
    if relu:
        o = jnp.where(o >= 0, o, 0.01 * o)
    o_ref[...] = o


def _dense_call(p, h, w0, w1, b2d):
    n, d = h.shape
    return pl.pallas_call(
        _dense_body,
        out_shape=(jax.ShapeDtypeStruct((n, d), jnp.float32),
                   jax.ShapeDtypeStruct((n, d), jnp.float32)),
    )(p, h, w0, w1, b2d)


def _finish_call(t, p, w2, relu):
    n, d = t.shape
    return pl.pallas_call(
        functools.partial(_finish_body, relu),
        out_shape=jax.ShapeDtypeStruct((n, d), jnp.float32),
    )(t, p, w2)


def kernel(x, edge_index, edge_attr, W0_0, W0_1, W0_2, b0,
           W1_0, W1_1, W1_2, b1, W2_0, W2_1, W2_2):
    n, d = x.shape
    e = edge_index.shape[1]
    src = edge_index[0]
    dst = edge_index[1]

    n_pad = -(-n // (_NS * 8)) * (_NS * 8)
    ept = -(-e // (_NW * _SUP)) * _SUP
    e_pad = ept * _NW
    pad = e_pad - e
    rt8 = e_pad // (8 * 128)
    srcp = jnp.concatenate([src, jnp.zeros((pad,), jnp.int32)]).reshape(rt8, 8, 128)
    pad_dst = jnp.arange(pad, dtype=jnp.int32) % jnp.int32(n_pad)
    dstp = jnp.concatenate([dst, pad_dst]).reshape(rt8, 8, 128)
    ewp = jnp.concatenate([edge_attr, jnp.zeros((pad,), jnp.float32)]).reshape(rt8, 8, 128)

    spmm = _make_spmm(n_pad, e_pad)

    def prop(cur):
        return spmm(cur, srcp, dstp, ewp).reshape(_NC, n_pad, d)

    def layer(h, w0, w1, w2, b2d, relu):
        p1 = prop(h)
        c1, t = _dense_call(p1, h, w0, w1, b2d)
        p2 = prop(c1)
        return _finish_call(t, p2, w2, relu)

    h = jnp.pad(x, ((0, n_pad - n), (0, 0)))
    h = layer(h, W0_0, W0_1, W0_2, b0.reshape(1, d), True)
    h = layer(h, W1_0, W1_1, W1_2, b1.reshape(1, d), True)
    w2p = [jnp.pad(w, ((0, 0), (0, d - w.shape[1]))) for w in (W2_0, W2_1, W2_2)]
    h = layer(h, w2p[0], w2p[1], w2p[2], jnp.zeros((1, d), jnp.float32), False)
    return h[:n, :W2_0.shape[1]]


# speedup vs baseline: 1.2297x; 1.0158x over previous
import functools

import jax
import jax.numpy as jnp
from jax import lax
from jax.experimental import pallas as pl
from jax.experimental.pallas import tpu as pltpu
from jax.experimental.pallas import tpu_sc as plsc

_NC = 2
_NS = 16
_NW = _NC * _NS
_D = 128
_SUP = 1024
_RPH = 2
_HALF = 256


def _chunks8(total, cap):
    out = []
    rem = total
    while rem > 0:
        c = min(cap, rem)
        out.append(c)
        rem -= c
    return out


@functools.lru_cache(maxsize=None)
def _make_spmm(n_pad, e_pad, d, dl):
    tot = e_pad // (_NS * _SUP)
    nsup0 = max(1, min(tot - 1, (tot * 7) // 10))
    nsup1 = tot - nsup0
    n_per_tile = n_pad // _NS
    zchunks = _chunks8(n_per_tile, _HALF)
    mesh = plsc.VectorSubcoreMesh(core_axis_name="c", subcore_axis_name="s",
                                  num_cores=_NC, num_subcores=_NS)

    @functools.partial(
        pl.kernel,
        out_type=jax.ShapeDtypeStruct((_NC * n_pad, d), jnp.float32),
        mesh=mesh,
        scratch_types=[
            pltpu.VMEM_SHARED((n_pad, d), jnp.float32),
            pltpu.VMEM((2, 8, 128), jnp.int32),
            pltpu.VMEM((2, 8, 128), jnp.int32),
            pltpu.VMEM((2, 8, 128), jnp.float32),
            pltpu.VMEM((_HALF, d), jnp.float32),
            pltpu.SemaphoreType.DMA,
            pltpu.SemaphoreType.DMA,
            pltpu.SemaphoreType.DMA,
            pltpu.SemaphoreType.DMA,
        ],
    )
    def spmm(cur_hbm, srcr_hbm, dstr_hbm, ewr_hbm, out_hbm,
             acc, src_v, dst_v, ew_v, rows_v, gsem, isem, ssem0, ssem1):
        cid = lax.axis_index("c")
        sid = lax.axis_index("s")
        nsup = jnp.where(cid == 0, nsup0, nsup1)
        qbase = sid * tot + cid * nsup0
        node_off = sid * n_per_tile

        pis = pltpu.async_copy(srcr_hbm.at[qbase], src_v.at[0], isem)
        pid = pltpu.async_copy(dstr_hbm.at[qbase], dst_v.at[0], isem)
        pie = pltpu.async_copy(ewr_hbm.at[qbase], ew_v.at[0], isem)

        def zrow(r, _):
            for j in range(d // 16):
                rows_v[r, pl.ds(j * 16, 16)] = jnp.zeros((16,), jnp.float32)
            return 0
        lax.fori_loop(0, _HALF, zrow, 0)
        zcs = []
        off = 0
        for zc in zchunks:
            zcs.append(pltpu.async_copy(rows_v.at[pl.ds(0, zc)],
                                        acc.at[pl.ds(node_off + off, zc)], gsem))
            off += zc
        for c in zcs:
            c.wait()
        plsc.subcore_barrier()

        pis.wait()
        pid.wait()
        pie.wait()
        pltpu.async_copy(cur_hbm.at[src_v.at[0, 0]],
                         rows_v.at[pl.ds(0, 128)], gsem)
        pltpu.async_copy(rows_v.at[pl.ds(128, 128)],
                         acc.at[dst_v.at[0, 0]], ssem1, add=True)

        def sup_body(s, _):
            s8 = lax.rem(s, 2)
            nxt = 1 - s8
            sn = qbase + jnp.minimum(s + 1, nsup - 1)
            dis = did = die = None
            for j in range(8):
                sl = (j % 2) * 128
                osl = 128 - sl
                psem = ssem1 if j % 2 == 0 else ssem0
                csem = ssem0 if j % 2 == 0 else ssem1
                pltpu.make_async_copy(cur_hbm.at[src_v.at[s8, j]],
                                      rows_v.at[pl.ds(sl, 128)], gsem).wait()
                pltpu.make_async_copy(cur_hbm.at[src_v.at[s8, j]],
                                      rows_v.at[pl.ds(osl, 128)], psem).wait()
                if j == 0:
                    dis = pltpu.async_copy(srcr_hbm.at[sn], src_v.at[nxt], isem)
                    did = pltpu.async_copy(dstr_hbm.at[sn], dst_v.at[nxt], isem)
                    die = pltpu.async_copy(ewr_hbm.at[sn], ew_v.at[nxt], isem)
                if j < 7:
                    pltpu.async_copy(cur_hbm.at[src_v.at[s8, j + 1]],
                                     rows_v.at[pl.ds(osl, 128)], gsem)
                else:
                    dis.wait()
                    did.wait()
                    die.wait()
                    pltpu.async_copy(cur_hbm.at[src_v.at[nxt, 0]],
                                     rows_v.at[pl.ds(0, 128)], gsem)

                def gbody(g, _, j=j, sl=sl, s8=s8):
                    wvec = ew_v[s8, j, pl.ds(g * 16, 16)]
                    for l in range(16):
                        eo = sl + g * 16 + l
                        w = wvec.at[jnp.full((16,), l, jnp.int32)].get(
                            mode="promise_in_bounds")
                        for q in range(dl // 16):
                            rows_v[eo, pl.ds(q * 16, 16)] = (
                                rows_v[eo, pl.ds(q * 16, 16)] * w)
                    return 0
                lax.fori_loop(0, 8, gbody, 0)

                pltpu.async_copy(rows_v.at[pl.ds(sl, 128)],
                                 acc.at[dst_v.at[s8, j]], csem, add=True)
            return 0
        lax.fori_loop(0, nsup, sup_body, 0)
        pltpu.make_async_copy(cur_hbm.at[src_v.at[0, 0]],
                              rows_v.at[pl.ds(0, 128)], gsem).wait()
        pltpu.make_async_copy(cur_hbm.at[src_v.at[0, 0]],
                              rows_v.at[pl.ds(128, 128)], ssem1).wait()
        plsc.subcore_barrier()

        out_row = cid * n_pad + node_off
        cps = []
        off = 0
        for zc in zchunks:
            cps.append(pltpu.async_copy(acc.at[pl.ds(node_off + off, zc)],
                                        out_hbm.at[pl.ds(out_row + off, zc)],
                                        gsem))
            off += zc
        for c in cps:
            c.wait()

    return spmm


def _dense_body(p_ref, h_ref, w0_ref, w1_ref, b_ref, c1_ref, t_ref):
    c1 = p_ref[0] + p_ref[1]
    c1_ref[...] = c1
    t_ref[...] = (jnp.dot(h_ref[...], w0_ref[...], preferred_element_type=jnp.float32)
                  + jnp.dot(c1, w1_ref[...], preferred_element_type=jnp.float32)
                  + b_ref[...])


def _finish_body(relu, t_ref, p_ref, w2_ref, o_ref):
    c2 = p_ref[0] + p_ref[1]
    o = t_ref[...] + jnp.dot(c2, w2_ref[...], preferred_element_type=jnp.float32)
    if relu:
        o = jnp.where(o >= 0, o, 0.01 * o)
    o_ref[...] = o


def _dense_call(p, h, w0, w1, b2d):
    n, d = h.shape
    return pl.pallas_call(
        _dense_body,
        out_shape=(jax.ShapeDtypeStruct((n, d), jnp.float32),
                   jax.ShapeDtypeStruct((n, d), jnp.float32)),
    )(p, h, w0, w1, b2d)


def _finish_call(t, p, w2, relu):
    n, d = t.shape
    return pl.pallas_call(
        functools.partial(_finish_body, relu),
        out_shape=jax.ShapeDtypeStruct((n, d), jnp.float32),
    )(t, p, w2)


def _l2pre_body(h_ref, w1_ref, w2_ref, a_ref, b_ref):
    a_ref[...] = jnp.dot(h_ref[...], w2_ref[...],
                         preferred_element_type=jnp.float32)
    b_ref[...] = jnp.dot(h_ref[...], w1_ref[...],
                         preferred_element_type=jnp.float32)


def _l2mid_body(p_ref, b_ref, o_ref):
    o_ref[...] = p_ref[0] + p_ref[1] + b_ref[...]


def _l2fin_body(h_ref, w0_ref, q_ref, o_ref):
    o_ref[...] = (jnp.dot(h_ref[...], w0_ref[...],
                          preferred_element_type=jnp.float32)
                  + q_ref[0] + q_ref[1])


def kernel(x, edge_index, edge_attr, W0_0, W0_1, W0_2, b0,
           W1_0, W1_1, W1_2, b1, W2_0, W2_1, W2_2):
    n, d = x.shape
    e = edge_index.shape[1]
    src = edge_index[0]
    dst = edge_index[1]

    n_pad = -(-n // (_NS * 8)) * (_NS * 8)
    ept = -(-e // (_NW * _SUP)) * _SUP
    e_pad = ept * _NW
    pad = e_pad - e
    rt8 = e_pad // (8 * 128)
    srcp = jnp.concatenate([src, jnp.zeros((pad,), jnp.int32)]).reshape(rt8, 8, 128)
    pad_dst = jnp.arange(pad, dtype=jnp.int32) % jnp.int32(n_pad)
    dstp = jnp.concatenate([dst, pad_dst]).reshape(rt8, 8, 128)
    ewp = jnp.concatenate([edge_attr, jnp.zeros((pad,), jnp.float32)]).reshape(rt8, 8, 128)

    spmm = _make_spmm(n_pad, e_pad, d, d)
    spmm_n = _make_spmm(n_pad, e_pad, d, 16)

    def prop(cur):
        return spmm(cur, srcp, dstp, ewp).reshape(_NC, n_pad, d)

    def prop_n(cur):
        return spmm_n(cur, srcp, dstp, ewp).reshape(_NC, n_pad, d)

    def layer(h, w0, w1, w2, b2d, relu):
        p1 = prop(h)
        c1, t = _dense_call(p1, h, w0, w1, b2d)
        p2 = prop(c1)
        return _finish_call(t, p2, w2, relu)

    h = jnp.pad(x, ((0, n_pad - n), (0, 0)))
    h = layer(h, W0_0, W0_1, W0_2, b0.reshape(1, d), True)
    h = layer(h, W1_0, W1_1, W1_2, b1.reshape(1, d), True)

    dw = d
    w2p = [jnp.pad(w, ((0, 0), (0, dw - w.shape[1])))
           for w in (W2_0, W2_1, W2_2)]
    a, bb = pl.pallas_call(
        _l2pre_body,
        out_shape=(jax.ShapeDtypeStruct((n_pad, dw), jnp.float32),
                   jax.ShapeDtypeStruct((n_pad, dw), jnp.float32)),
    )(h, w2p[1], w2p[2])
    p = prop_n(a)
    in2 = pl.pallas_call(
        _l2mid_body,
        out_shape=jax.ShapeDtypeStruct((n_pad, dw), jnp.float32),
    )(p, bb)
    q = prop_n(in2)
    o = pl.pallas_call(
        _l2fin_body,
        out_shape=jax.ShapeDtypeStruct((n_pad, dw), jnp.float32),
    )(h, w2p[0], q)
    return o[:n, :W2_0.shape[1]]


# speedup vs baseline: 1.3542x; 1.1012x over previous
import functools

import jax
import jax.numpy as jnp
from jax import lax
from jax.experimental import pallas as pl
from jax.experimental.pallas import tpu as pltpu
from jax.experimental.pallas import tpu_sc as plsc

_NC = 2
_NS = 16
_NW = _NC * _NS
_D = 128
_SUP = 1024
_RPH = 2
_HALF = 256


def _chunks8(total, cap):
    out = []
    rem = total
    while rem > 0:
        c = min(cap, rem)
        out.append(c)
        rem -= c
    return out


@functools.lru_cache(maxsize=None)
def _make_spmm(n_pad, e_pad, d, dl):
    tot = e_pad // (_NS * _SUP)
    nsup0 = max(1, min(tot - 1, (tot * 3) // 4))
    nsup1 = tot - nsup0
    n_per_tile = n_pad // _NS
    zchunks = _chunks8(n_per_tile, _HALF)
    mesh = plsc.VectorSubcoreMesh(core_axis_name="c", subcore_axis_name="s",
                                  num_cores=_NC, num_subcores=_NS)

    @functools.partial(
        pl.kernel,
        out_type=jax.ShapeDtypeStruct((_NC * n_pad, d), jnp.float32),
        mesh=mesh,
        scratch_types=[
            pltpu.VMEM_SHARED((n_pad, d), jnp.float32),
            pltpu.VMEM((2, 8, 128), jnp.int32),
            pltpu.VMEM((2, 8, 128), jnp.int32),
            pltpu.VMEM((2, 8, 128), jnp.float32),
            pltpu.VMEM((_HALF, d), jnp.float32),
            pltpu.SemaphoreType.DMA,
            pltpu.SemaphoreType.DMA,
            pltpu.SemaphoreType.DMA,
            pltpu.SemaphoreType.DMA,
        ],
    )
    def spmm(cur_hbm, srcr_hbm, dstr_hbm, ewr_hbm, out_hbm,
             acc, src_v, dst_v, ew_v, rows_v, gsem, isem, ssem0, ssem1):
        cid = lax.axis_index("c")
        sid = lax.axis_index("s")
        nsup = jnp.where(cid == 0, nsup0, nsup1)
        qbase = sid * tot + cid * nsup0
        node_off = sid * n_per_tile

        pis = pltpu.async_copy(srcr_hbm.at[qbase], src_v.at[0], isem)
        pid = pltpu.async_copy(dstr_hbm.at[qbase], dst_v.at[0], isem)
        pie = pltpu.async_copy(ewr_hbm.at[qbase], ew_v.at[0], isem)

        def zrow(r, _):
            for j in range(d // 16):
                rows_v[r, pl.ds(j * 16, 16)] = jnp.zeros((16,), jnp.float32)
            return 0
        lax.fori_loop(0, _HALF, zrow, 0)
        zcs = []
        off = 0
        for zc in zchunks:
            zcs.append(pltpu.async_copy(rows_v.at[pl.ds(0, zc)],
                                        acc.at[pl.ds(node_off + off, zc)], gsem))
            off += zc
        for c in zcs:
            c.wait()
        plsc.subcore_barrier()

        pis.wait()
        pid.wait()
        pie.wait()
        pltpu.async_copy(cur_hbm.at[src_v.at[0, 0]],
                         rows_v.at[pl.ds(0, 128)], gsem)
        pltpu.async_copy(rows_v.at[pl.ds(128, 128)],
                         acc.at[dst_v.at[0, 0]], ssem1, add=True)

        def sup_body(s, _):
            s8 = lax.rem(s, 2)
            nxt = 1 - s8
            sn = qbase + jnp.minimum(s + 1, nsup - 1)
            dis = did = die = None
            for j in range(8):
                sl = (j % 2) * 128
                osl = 128 - sl
                psem = ssem1 if j % 2 == 0 else ssem0
                csem = ssem0 if j % 2 == 0 else ssem1
                pltpu.make_async_copy(cur_hbm.at[src_v.at[s8, j]],
                                      rows_v.at[pl.ds(sl, 128)], gsem).wait()
                pltpu.make_async_copy(cur_hbm.at[src_v.at[s8, j]],
                                      rows_v.at[pl.ds(osl, 128)], psem).wait()
                if j == 0:
                    dis = pltpu.async_copy(srcr_hbm.at[sn], src_v.at[nxt], isem)
                    did = pltpu.async_copy(dstr_hbm.at[sn], dst_v.at[nxt], isem)
                    die = pltpu.async_copy(ewr_hbm.at[sn], ew_v.at[nxt], isem)
                if j < 7:
                    pltpu.async_copy(cur_hbm.at[src_v.at[s8, j + 1]],
                                     rows_v.at[pl.ds(osl, 128)], gsem)
                else:
                    dis.wait()
                    did.wait()
                    die.wait()
                    pltpu.async_copy(cur_hbm.at[src_v.at[nxt, 0]],
                                     rows_v.at[pl.ds(0, 128)], gsem)

                def gbody(g, _, j=j, sl=sl, s8=s8):
                    wvec = ew_v[s8, j, pl.ds(g * 16, 16)]
                    for l in range(16):
                        eo = sl + g * 16 + l
                        w = wvec.at[jnp.full((16,), l, jnp.int32)].get(
                            mode="promise_in_bounds")
                        for q in range(dl // 16):
                            rows_v[eo, pl.ds(q * 16, 16)] = (
                                rows_v[eo, pl.ds(q * 16, 16)] * w)
                    return 0
                lax.fori_loop(0, 8, gbody, 0)

                pltpu.async_copy(rows_v.at[pl.ds(sl, 128)],
                                 acc.at[dst_v.at[s8, j]], csem, add=True)
            return 0
        lax.fori_loop(0, nsup, sup_body, 0)
        pltpu.make_async_copy(cur_hbm.at[src_v.at[0, 0]],
                              rows_v.at[pl.ds(0, 128)], gsem).wait()
        pltpu.make_async_copy(cur_hbm.at[src_v.at[0, 0]],
                              rows_v.at[pl.ds(128, 128)], ssem1).wait()
        plsc.subcore_barrier()

        out_row = cid * n_pad + node_off
        cps = []
        off = 0
        for zc in zchunks:
            cps.append(pltpu.async_copy(acc.at[pl.ds(node_off + off, zc)],
                                        out_hbm.at[pl.ds(out_row + off, zc)],
                                        gsem))
            off += zc
        for c in cps:
            c.wait()

    return spmm


def _dense_body(p_ref, h_ref, w0_ref, w1_ref, b_ref, c1_ref, t_ref):
    c1 = p_ref[0] + p_ref[1]
    c1_ref[...] = c1
    t_ref[...] = (jnp.dot(h_ref[...], w0_ref[...], preferred_element_type=jnp.float32)
                  + jnp.dot(c1, w1_ref[...], preferred_element_type=jnp.float32)
                  + b_ref[...])


def _finish_body(relu, t_ref, p_ref, w2_ref, o_ref):
    c2 = p_ref[0] + p_ref[1]
    o = t_ref[...] + jnp.dot(c2, w2_ref[...], preferred_element_type=jnp.float32)
    if relu:
        o = jnp.where(o >= 0, o, 0.01 * o)
    o_ref[...] = o


def _dense_call(p, h, w0, w1, b2d):
    n, d = h.shape
    return pl.pallas_call(
        _dense_body,
        out_shape=(jax.ShapeDtypeStruct((n, d), jnp.float32),
                   jax.ShapeDtypeStruct((n, d), jnp.float32)),
    )(p, h, w0, w1, b2d)


def _finish_call(t, p, w2, relu):
    n, d = t.shape
    return pl.pallas_call(
        functools.partial(_finish_body, relu),
        out_shape=jax.ShapeDtypeStruct((n, d), jnp.float32),
    )(t, p, w2)


def _l2pre_body(h_ref, w1_ref, w2_ref, a_ref, b_ref):
    a_ref[...] = jnp.dot(h_ref[...], w2_ref[...],
                         preferred_element_type=jnp.float32)
    b_ref[...] = jnp.dot(h_ref[...], w1_ref[...],
                         preferred_element_type=jnp.float32)


def _l2mid_body(p_ref, b_ref, o_ref):
    o_ref[...] = p_ref[0] + p_ref[1] + b_ref[...]


def _l2fin_body(h_ref, w0_ref, q_ref, o_ref):
    o_ref[...] = (jnp.dot(h_ref[...], w0_ref[...],
                          preferred_element_type=jnp.float32)
                  + q_ref[0] + q_ref[1])


def kernel(x, edge_index, edge_attr, W0_0, W0_1, W0_2, b0,
           W1_0, W1_1, W1_2, b1, W2_0, W2_1, W2_2):
    n, d = x.shape
    e = edge_index.shape[1]
    src = edge_index[0]
    dst = edge_index[1]

    n_pad = -(-n // (_NS * 8)) * (_NS * 8)
    ept = -(-e // (_NW * _SUP)) * _SUP
    e_pad = ept * _NW
    pad = e_pad - e
    rt8 = e_pad // (8 * 128)
    srcp = jnp.concatenate([src, jnp.zeros((pad,), jnp.int32)]).reshape(rt8, 8, 128)
    pad_dst = jnp.arange(pad, dtype=jnp.int32) % jnp.int32(n_pad)
    dstp = jnp.concatenate([dst, pad_dst]).reshape(rt8, 8, 128)
    ewp = jnp.concatenate([edge_attr, jnp.zeros((pad,), jnp.float32)]).reshape(rt8, 8, 128)

    spmm = _make_spmm(n_pad, e_pad, d, d)
    spmm_n = _make_spmm(n_pad, e_pad, d, 16)

    def prop(cur):
        return spmm(cur, srcp, dstp, ewp).reshape(_NC, n_pad, d)

    def prop_n(cur):
        return spmm_n(cur, srcp, dstp, ewp).reshape(_NC, n_pad, d)

    def layer(h, w0, w1, w2, b2d, relu):
        p1 = prop(h)
        c1, t = _dense_call(p1, h, w0, w1, b2d)
        p2 = prop(c1)
        return _finish_call(t, p2, w2, relu)

    h = jnp.pad(x, ((0, n_pad - n), (0, 0)))
    h = layer(h, W0_0, W0_1, W0_2, b0.reshape(1, d), True)
    h = layer(h, W1_0, W1_1, W1_2, b1.reshape(1, d), True)

    dw = d
    w2p = [jnp.pad(w, ((0, 0), (0, dw - w.shape[1])))
           for w in (W2_0, W2_1, W2_2)]
    a, bb = pl.pallas_call(
        _l2pre_body,
        out_shape=(jax.ShapeDtypeStruct((n_pad, dw), jnp.float32),
                   jax.ShapeDtypeStruct((n_pad, dw), jnp.float32)),
    )(h, w2p[1], w2p[2])
    p = prop_n(a)
    in2 = pl.pallas_call(
        _l2mid_body,
        out_shape=jax.ShapeDtypeStruct((n_pad, dw), jnp.float32),
    )(p, bb)
    q = prop_n(in2)
    o = pl.pallas_call(
        _l2fin_body,
        out_shape=jax.ShapeDtypeStruct((n_pad, dw), jnp.float32),
    )(h, w2p[0], q)
    return o[:n, :W2_0.shape[1]]
